# baseline (speedup 1.0000x reference)
import functools

import jax
import jax.numpy as jnp
from jax import lax
from jax.experimental import pallas as pl
from jax.experimental.pallas import tpu as pltpu
from jax.experimental.pallas import tpu_sc as plsc

N = 20000
G = 8
D = 2048
NUM_CLASSES = 30
C_MM = NUM_CLASSES + 1 + NUM_CLASSES * 4
C_TAIL = NUM_CLASSES + 1 + 4
C_OUT = C_MM + C_TAIL

NW = 32
ROWS_W = 640
NPAD = NW * ROWS_W
NGROUPS = ROWS_W // 16

BN = 1000

_LN2 = 0.6931471805599453
_SQRT2 = 1.4142135623730951


def _ln(x):
    bits = lax.bitcast_convert_type(x, jnp.int32)
    e = ((bits >> 23) & 0xFF) - 127
    m = lax.bitcast_convert_type((bits & 0x007FFFFF) | 0x3F800000,
                                 jnp.float32)
    big = m > _SQRT2
    m = jnp.where(big, m * 0.5, m)
    e = e + jnp.where(big, 1, 0)
    r = (m - 1.0) / (m + 1.0)
    r2 = r * r
    s = r * (2.0 + r2 * (2.0 / 3.0 + r2 * (2.0 / 5.0)))
    return e.astype(jnp.float32) * _LN2 + s


def _splat_i32(v, n=16):
    return jnp.full((n,), v, jnp.int32)


def _sc_body(prop_hbm, gt_hbm, lab_hbm, out_hbm, pv, gv, lv, ov):
    wid = lax.axis_index("s") * 2 + lax.axis_index("c")
    base = jnp.where(wid == NW - 1, N - ROWS_W, wid * ROWS_W)
    pltpu.sync_copy(prop_hbm.at[pl.ds(base * 4, ROWS_W * 4)], pv)
    pltpu.sync_copy(gt_hbm, gv)
    pltpu.sync_copy(lab_hbm, lv)

    gx1 = [plsc.load_gather(gv, [_splat_i32(8 + 4 * g + 0)]) for g in range(G)]
    gy1 = [plsc.load_gather(gv, [_splat_i32(8 + 4 * g + 1)]) for g in range(G)]
    gx2 = [plsc.load_gather(gv, [_splat_i32(8 + 4 * g + 2)]) for g in range(G)]
    gy2 = [plsc.load_gather(gv, [_splat_i32(8 + 4 * g + 3)]) for g in range(G)]
    lab = [plsc.load_gather(lv, [_splat_i32(8 + g)]) for g in range(G)]
    area_g = [(gx2[g] - gx1[g] + 1.0) * (gy2[g] - gy1[g] + 1.0)
              for g in range(G)]

    lane = lax.iota(jnp.int32, 16)
    zero16 = jnp.zeros((16,), jnp.float32)

    def group(j, _):
        r = j * 16
        rows = r + lane
        p4 = rows * 4
        px1 = plsc.load_gather(pv, [p4 + 0])
        py1 = plsc.load_gather(pv, [p4 + 1])
        px2 = plsc.load_gather(pv, [p4 + 2])
        py2 = plsc.load_gather(pv, [p4 + 3])
        area_b = (px2 - px1 + 1.0) * (py2 - py1 + 1.0)

        for k in range(C_TAIL):
            ov[pl.ds(r * C_TAIL + 16 * k, 16)] = zero16

        row35 = rows * C_TAIL
        best = jnp.full((16,), -1.0, jnp.float32)
        bx1 = zero16
        by1 = zero16
        bx2 = zero16
        by2 = zero16
        for g in range(G):
            iw = jnp.maximum(
                jnp.minimum(px2, gx2[g]) - jnp.maximum(px1, gx1[g]) + 1.0,
                0.0)
            ih = jnp.maximum(
                jnp.minimum(py2, gy2[g]) - jnp.maximum(py1, gy1[g]) + 1.0,
                0.0)
            inter = iw * ih
            union = area_b + area_g[g] - inter
            iou = inter / jnp.maximum(union, 1e-6)

            idx = row35 + lab[g]
            cur = plsc.load_gather(ov, [idx])
            plsc.store_scatter(ov, [idx], jnp.maximum(cur, iou))

            upd = iou > best
            best = jnp.where(upd, iou, best)
            bx1 = jnp.where(upd, gx1[g], bx1)
            by1 = jnp.where(upd, gy1[g], by1)
            bx2 = jnp.where(upd, gx2[g], bx2)
            by2 = jnp.where(upd, gy2[g], by2)

        src_w = jnp.maximum(px2 - px1, 1e-3)
        src_h = jnp.maximum(py2 - py1, 1e-3)
        gt_w = jnp.maximum(bx2 - bx1, 1e-3)
        gt_h = jnp.maximum(by2 - by1, 1e-3)
        tx = (bx1 + 0.5 * gt_w - px1 - 0.5 * src_w) / src_w
        ty = (by1 + 0.5 * gt_h - py1 - 0.5 * src_h) / src_h

        plsc.store_scatter(ov, [row35 + NUM_CLASSES], best)
        plsc.store_scatter(ov, [row35 + (NUM_CLASSES + 1)], tx)
        plsc.store_scatter(ov, [row35 + (NUM_CLASSES + 2)], ty)
        plsc.store_scatter(ov, [row35 + (NUM_CLASSES + 3)], _ln(gt_w / src_w))
        plsc.store_scatter(ov, [row35 + (NUM_CLASSES + 4)], _ln(gt_h / src_h))
        return 0

    lax.fori_loop(0, NGROUPS, group, 0)
    pltpu.sync_copy(ov, out_hbm.at[pl.ds(base * C_TAIL, ROWS_W * C_TAIL)])


_sc_tail = functools.partial(
    pl.kernel,
    out_type=jax.ShapeDtypeStruct((N * C_TAIL,), jnp.float32),
    mesh=plsc.VectorSubcoreMesh(core_axis_name="c", subcore_axis_name="s"),
    scratch_types=[
        pltpu.VMEM((ROWS_W * 4,), jnp.float32),
        pltpu.VMEM((8 + 4 * G,), jnp.float32),
        pltpu.VMEM((16,), jnp.int32),
        pltpu.VMEM((ROWS_W * C_TAIL,), jnp.float32),
    ],
    compiler_params=pltpu.CompilerParams(needs_layout_passes=False),
)(_sc_body)


def _mm_body(x_ref, wc_ref, wb_ref, t_ref, out_ref):
    xv = x_ref[...]
    mm1 = jnp.dot(xv, wc_ref[...], preferred_element_type=jnp.float32)
    mm2 = jnp.dot(xv, wb_ref[...], preferred_element_type=jnp.float32)
    out_ref[...] = jnp.concatenate([mm1, mm2, t_ref[...]], axis=1)


def _fuse(x, wc, wb, tail):
    return pl.pallas_call(
        _mm_body,
        grid=(N // BN,),
        in_specs=[
            pl.BlockSpec((BN, D), lambda i: (i, 0)),
            pl.BlockSpec((D, NUM_CLASSES + 1), lambda i: (0, 0)),
            pl.BlockSpec((D, NUM_CLASSES * 4), lambda i: (0, 0)),
            pl.BlockSpec((BN, C_TAIL), lambda i: (i, 0)),
        ],
        out_specs=pl.BlockSpec((BN, C_OUT), lambda i: (i, 0)),
        out_shape=jax.ShapeDtypeStruct((N, C_OUT), jnp.float32),
        compiler_params=pltpu.CompilerParams(
            dimension_semantics=("parallel",),
        ),
    )(x, wc, wb, tail)


def kernel(x, proposals, gt_bbox, W_cls, W_bbox, gt_labels):
    prop_flat = proposals.reshape(N * 4)
    gt_flat = jnp.concatenate(
        [jnp.zeros((8,), jnp.float32), gt_bbox.reshape(4 * G)])
    lab16 = jnp.pad(gt_labels.astype(jnp.int32), (8, 16 - G - 8))

    tail = _sc_tail(prop_flat, gt_flat, lab16).reshape(N, C_TAIL)
    return _fuse(x, W_cls, W_bbox, tail)


# speedup vs baseline: 1.0450x; 1.0450x over previous
import functools

import jax
import jax.numpy as jnp
from jax import lax
from jax.experimental import pallas as pl
from jax.experimental.pallas import tpu as pltpu
from jax.experimental.pallas import tpu_sc as plsc

N = 20000
G = 8
D = 2048
NUM_CLASSES = 30
C_MM = NUM_CLASSES + 1 + NUM_CLASSES * 4
C_TAIL = NUM_CLASSES + 1 + 4
C_OUT = C_MM + C_TAIL

NW = 32
ROWS_W = 640
NPAD = NW * ROWS_W
NGROUPS = ROWS_W // 16

BN = 1000

_LN2 = 0.6931471805599453
_SQRT2 = 1.4142135623730951


def _ln(x):
    bits = lax.bitcast_convert_type(x, jnp.int32)
    e = ((bits >> 23) & 0xFF) - 127
    m = lax.bitcast_convert_type((bits & 0x007FFFFF) | 0x3F800000,
                                 jnp.float32)
    big = m > _SQRT2
    m = jnp.where(big, m * 0.5, m)
    e = e + jnp.where(big, 1, 0)
    r = (m - 1.0) / (m + 1.0)
    r2 = r * r
    s = r * (2.0 + r2 * (2.0 / 3.0 + r2 * (2.0 / 5.0)))
    return e.astype(jnp.float32) * _LN2 + s


def _splat_i32(v, n=16):
    return jnp.full((n,), v, jnp.int32)


def _sc_body(prop_hbm, gt_hbm, lab_hbm, out_hbm, pv, gv, lv, ov):
    wid = lax.axis_index("s") * 2 + lax.axis_index("c")
    base = jnp.where(wid == NW - 1, N - ROWS_W, wid * ROWS_W)
    pltpu.sync_copy(prop_hbm.at[pl.ds(base * 4, ROWS_W * 4)], pv)
    pltpu.sync_copy(gt_hbm, gv)
    pltpu.sync_copy(lab_hbm, lv)

    gx1 = [plsc.load_gather(gv, [_splat_i32(8 + 4 * g + 0)]) for g in range(G)]
    gy1 = [plsc.load_gather(gv, [_splat_i32(8 + 4 * g + 1)]) for g in range(G)]
    gx2 = [plsc.load_gather(gv, [_splat_i32(8 + 4 * g + 2)]) for g in range(G)]
    gy2 = [plsc.load_gather(gv, [_splat_i32(8 + 4 * g + 3)]) for g in range(G)]
    lab = [plsc.load_gather(lv, [_splat_i32(8 + g)]) for g in range(G)]
    area_g = [(gx2[g] - gx1[g] + 1.0) * (gy2[g] - gy1[g] + 1.0)
              for g in range(G)]

    lane = lax.iota(jnp.int32, 16)
    zero16 = jnp.zeros((16,), jnp.float32)

    def group(j, _):
        r = j * 16
        rows = r + lane
        p4 = rows * 4
        px1 = plsc.load_gather(pv, [p4 + 0])
        py1 = plsc.load_gather(pv, [p4 + 1])
        px2 = plsc.load_gather(pv, [p4 + 2])
        py2 = plsc.load_gather(pv, [p4 + 3])
        area_b = (px2 - px1 + 1.0) * (py2 - py1 + 1.0)

        for k in range(C_TAIL):
            ov[pl.ds(r * C_TAIL + 16 * k, 16)] = zero16

        row35 = rows * C_TAIL
        best = jnp.full((16,), -1.0, jnp.float32)
        bx1 = zero16
        by1 = zero16
        bx2 = zero16
        by2 = zero16
        for g in range(G):
            iw = jnp.maximum(
                jnp.minimum(px2, gx2[g]) - jnp.maximum(px1, gx1[g]) + 1.0,
                0.0)
            ih = jnp.maximum(
                jnp.minimum(py2, gy2[g]) - jnp.maximum(py1, gy1[g]) + 1.0,
                0.0)
            inter = iw * ih
            union = area_b + area_g[g] - inter
            iou = inter / jnp.maximum(union, 1e-6)

            idx = row35 + lab[g]
            cur = plsc.load_gather(ov, [idx])
            plsc.store_scatter(ov, [idx], jnp.maximum(cur, iou))

            upd = iou > best
            best = jnp.where(upd, iou, best)
            bx1 = jnp.where(upd, gx1[g], bx1)
            by1 = jnp.where(upd, gy1[g], by1)
            bx2 = jnp.where(upd, gx2[g], bx2)
            by2 = jnp.where(upd, gy2[g], by2)

        src_w = jnp.maximum(px2 - px1, 1e-3)
        src_h = jnp.maximum(py2 - py1, 1e-3)
        gt_w = jnp.maximum(bx2 - bx1, 1e-3)
        gt_h = jnp.maximum(by2 - by1, 1e-3)
        tx = (bx1 + 0.5 * gt_w - px1 - 0.5 * src_w) / src_w
        ty = (by1 + 0.5 * gt_h - py1 - 0.5 * src_h) / src_h

        plsc.store_scatter(ov, [row35 + NUM_CLASSES], best)
        plsc.store_scatter(ov, [row35 + (NUM_CLASSES + 1)], tx)
        plsc.store_scatter(ov, [row35 + (NUM_CLASSES + 2)], ty)
        plsc.store_scatter(ov, [row35 + (NUM_CLASSES + 3)], _ln(gt_w / src_w))
        plsc.store_scatter(ov, [row35 + (NUM_CLASSES + 4)], _ln(gt_h / src_h))
        return 0

    lax.fori_loop(0, NGROUPS, group, 0)
    pltpu.sync_copy(ov, out_hbm.at[pl.ds(base * C_TAIL, ROWS_W * C_TAIL)])


_sc_tail = functools.partial(
    pl.kernel,
    out_type=jax.ShapeDtypeStruct((N * C_TAIL,), jnp.float32),
    mesh=plsc.VectorSubcoreMesh(core_axis_name="c", subcore_axis_name="s"),
    scratch_types=[
        pltpu.VMEM((ROWS_W * 4,), jnp.float32),
        pltpu.VMEM((8 + 4 * G,), jnp.float32),
        pltpu.VMEM((16,), jnp.int32),
        pltpu.VMEM((ROWS_W * C_TAIL,), jnp.float32),
    ],
    compiler_params=pltpu.CompilerParams(needs_layout_passes=False),
)(_sc_body)


def _mm_body(x_ref, w_ref, t_ref, out_ref):
    mm = jnp.dot(x_ref[...], w_ref[...],
                 preferred_element_type=jnp.float32)
    out_ref[...] = jnp.concatenate([mm, t_ref[...]], axis=1)


def _fuse(x, w, tail):
    return pl.pallas_call(
        _mm_body,
        grid=(N // BN,),
        in_specs=[
            pl.BlockSpec((BN, D), lambda i: (i, 0)),
            pl.BlockSpec((D, C_MM), lambda i: (0, 0)),
            pl.BlockSpec((BN, C_TAIL), lambda i: (i, 0)),
        ],
        out_specs=pl.BlockSpec((BN, C_OUT), lambda i: (i, 0)),
        out_shape=jax.ShapeDtypeStruct((N, C_OUT), jnp.float32),
        compiler_params=pltpu.CompilerParams(
            dimension_semantics=("parallel",),
        ),
    )(x, w, tail)


def kernel(x, proposals, gt_bbox, W_cls, W_bbox, gt_labels):
    w = jnp.concatenate([W_cls, W_bbox], axis=1)
    prop_flat = proposals.reshape(N * 4)
    gt_flat = jnp.concatenate(
        [jnp.zeros((8,), jnp.float32), gt_bbox.reshape(4 * G)])
    lab16 = jnp.pad(gt_labels.astype(jnp.int32), (8, 16 - G - 8))

    tail = _sc_tail(prop_flat, gt_flat, lab16).reshape(N, C_TAIL)
    return _fuse(x, w, tail)
